# baseline (speedup 1.0000x reference)
import functools

import jax
import jax.numpy as jnp
from jax import lax
from jax.experimental import pallas as pl
from jax.experimental.pallas import tpu as pltpu
from jax.experimental.pallas import tpu_sc as plsc

VOCAB = 1000000
D = 64
T_LEN = 200
B = 4096
B_TOTAL = B * T_LEN

NC = 2
NS = 16
NW = NC * NS

CHUNK = 128
PER_W = B_TOTAL // NW
N_CHUNKS = PER_W // CHUNK
BLKS = B // CHUNK

NBUF = 4
WBUF = 2


def _make_kernel():
  mesh = plsc.VectorSubcoreMesh(core_axis_name="c", subcore_axis_name="s")

  @functools.partial(
      pl.kernel,
      mesh=mesh,
      compiler_params=pltpu.CompilerParams(
          use_tc_tiling_on_sc=False, needs_layout_passes=False),
      out_type=jax.ShapeDtypeStruct((T_LEN, 8, BLKS, 8, CHUNK), jnp.float32),
      scratch_types=[
          pltpu.VMEM((N_CHUNKS, CHUNK), jnp.int32),
          pltpu.VMEM((NBUF, CHUNK, D), jnp.float32),
          pltpu.VMEM((WBUF, 8, 8, CHUNK), jnp.float32),
          pltpu.SemaphoreType.DMA((NBUF,)),
          pltpu.SemaphoreType.DMA((WBUF,)),
      ],
  )
  def emb_kernel(idx_hbm, table_hbm, out_hbm, idx_v, rows_v, tp_v, gsem, wsem):
    wid = lax.axis_index("s") * NC + lax.axis_index("c")
    pair0 = wid * N_CHUNKS

    pltpu.sync_copy(idx_hbm.at[wid], idx_v)

    lane = lax.iota(jnp.int32, 16)
    b_idx = [lane + (16 * b0) for b0 in range(8)]

    def gather(j, gb):
      return pltpu.make_async_copy(
          table_hbm.at[idx_v.at[j]], rows_v.at[gb], gsem.at[gb])

    def write(j, wb):
      q = pair0 + j
      t = (q // 256) * 8 + lax.rem(q, 8)
      blk = lax.rem(q // 8, BLKS)
      return pltpu.make_async_copy(
          tp_v.at[wb], out_hbm.at[t, :, blk], wsem.at[wb])

    def transpose(gb, wb):
      rows = rows_v.at[gb]
      tp = tp_v.at[wb]

      def dbody(d, _):
        d_idx = jnp.full((16,), d, jnp.int32)
        d_hi = d // 8
        d_lo = lax.rem(d, 8)
        for b0 in range(8):
          v = plsc.load_gather(rows, [b_idx[b0], d_idx])
          tp[d_hi, d_lo, pl.ds(16 * b0, 16)] = v
        return 0

      lax.fori_loop(0, D, dbody, 0)

    def step(j, gb, wb, first, last):
      gather(j, gb).wait()
      if not first:
        write(j - WBUF, wb).wait()
      transpose(gb, wb)
      write(j, wb).start()
      if last is None:
        jn = j + NBUF

        @pl.when(jn < N_CHUNKS)
        def _():
          gather(jn, gb).start()
      elif not last:
        gather(j + NBUF, gb).start()

    for gb in range(NBUF):
      gather(gb, gb).start()
    for j in range(NBUF):
      step(j, j % NBUF, j % WBUF, first=(j < WBUF), last=False)

    def body(i, _):
      for u in range(NBUF):
        j = i * NBUF + u
        step(j, u, j % WBUF, first=False, last=None)
      return 0

    lax.fori_loop(1, N_CHUNKS // NBUF, body, 0)

    for j in range(N_CHUNKS - WBUF, N_CHUNKS):
      write(j, j % WBUF).wait()

  return emb_kernel


_emb = _make_kernel()


@jax.jit
def kernel(x, weight):
  idx = (x.astype(jnp.int32).reshape(BLKS, CHUNK, T_LEN // 8, 8)
         .transpose(2, 0, 3, 1).reshape(NW, N_CHUNKS, CHUNK))
  y = _emb(idx, weight)
  out = y.transpose(2, 4, 0, 1, 3).reshape(B, T_LEN, D)
  return out


# speedup vs baseline: 1.8354x; 1.8354x over previous
import functools

import jax
import jax.numpy as jnp
from jax import lax
from jax.experimental import pallas as pl
from jax.experimental.pallas import tpu as pltpu
from jax.experimental.pallas import tpu_sc as plsc

VOCAB = 1000000
D = 64
T_LEN = 200
B = 4096
B_TOTAL = B * T_LEN

NC = 2
NS = 16
NW = NC * NS

CHUNK = 128
PER_W = B_TOTAL // NW
N_CHUNKS = PER_W // CHUNK
BLKS = B // CHUNK

NBUF = 4
WBUF = 2


def _make_kernel():
  mesh = plsc.VectorSubcoreMesh(core_axis_name="c", subcore_axis_name="s")

  @functools.partial(
      pl.kernel,
      mesh=mesh,
      compiler_params=pltpu.CompilerParams(
          use_tc_tiling_on_sc=False, needs_layout_passes=False),
      out_type=jax.ShapeDtypeStruct((T_LEN, 8, BLKS, 8, CHUNK), jnp.float32),
      scratch_types=[
          pltpu.VMEM((N_CHUNKS, CHUNK), jnp.int32),
          pltpu.VMEM((NBUF, CHUNK, D), jnp.float32),
          pltpu.VMEM((WBUF, 8, 8, CHUNK + 1), jnp.float32),
          pltpu.SemaphoreType.DMA((NBUF,)),
          pltpu.SemaphoreType.DMA((WBUF,)),
      ],
  )
  def emb_kernel(idx_hbm, table_hbm, out_hbm, idx_v, rows_v, tp_v, gsem, wsem):
    wid = lax.axis_index("s") * NC + lax.axis_index("c")
    pair0 = wid * N_CHUNKS

    pltpu.sync_copy(idx_hbm.at[wid], idx_v)

    lane = lax.iota(jnp.int32, 16)
    dlo_c = lax.rem(lane, 8)
    dhi_c = [2 * d0 + lane // 8 for d0 in range(4)]

    def gather(j, gb):
      return pltpu.make_async_copy(
          table_hbm.at[idx_v.at[j]], rows_v.at[gb], gsem.at[gb])

    def write(j, wb):
      q = pair0 + j
      t = (q // 256) * 8 + lax.rem(q, 8)
      blk = lax.rem(q // 8, BLKS)
      return pltpu.make_async_copy(
          tp_v.at[wb, :, :, pl.ds(0, CHUNK)], out_hbm.at[t, :, blk],
          wsem.at[wb])

    def transpose(gb, wb):
      rows = rows_v.at[gb]
      tp = tp_v.at[wb]

      def bbody(i, _):
        for k in range(2):
          b = 2 * i + k
          bv = jnp.full((16,), b, jnp.int32)
          for d0 in range(4):
            v = rows[b, pl.ds(16 * d0, 16)]
            plsc.store_scatter(tp, [dhi_c[d0], dlo_c, bv], v)
        return 0

      lax.fori_loop(0, CHUNK // 2, bbody, 0)

    def step(j, gb, wb, first, last):
      gather(j, gb).wait()
      if not first:
        write(j - WBUF, wb).wait()
      transpose(gb, wb)
      write(j, wb).start()
      if last is None:
        jn = j + NBUF

        @pl.when(jn < N_CHUNKS)
        def _():
          gather(jn, gb).start()
      elif not last:
        gather(j + NBUF, gb).start()

    for gb in range(NBUF):
      gather(gb, gb).start()
    for j in range(NBUF):
      step(j, j % NBUF, j % WBUF, first=(j < WBUF), last=False)

    def body(i, _):
      for u in range(NBUF):
        j = i * NBUF + u
        step(j, u, j % WBUF, first=False, last=None)
      return 0

    lax.fori_loop(1, N_CHUNKS // NBUF, body, 0)

    for j in range(N_CHUNKS - WBUF, N_CHUNKS):
      write(j, j % WBUF).wait()

  return emb_kernel


_emb = _make_kernel()


@jax.jit
def kernel(x, weight):
  idx = (x.astype(jnp.int32).reshape(BLKS, CHUNK, T_LEN // 8, 8)
         .transpose(2, 0, 3, 1).reshape(NW, N_CHUNKS, CHUNK))
  y = _emb(idx, weight)
  out = y.transpose(2, 4, 0, 1, 3).reshape(B, T_LEN, D)
  return out


# speedup vs baseline: 1.8358x; 1.0002x over previous
import functools

import jax
import jax.numpy as jnp
from jax import lax
from jax.experimental import pallas as pl
from jax.experimental.pallas import tpu as pltpu
from jax.experimental.pallas import tpu_sc as plsc

VOCAB = 1000000
D = 64
T_LEN = 200
B = 4096
B_TOTAL = B * T_LEN

NC = 2
NS = 16
NW = NC * NS

CHUNK = 128
PER_W = B_TOTAL // NW
N_CHUNKS = PER_W // CHUNK
BLKS = B // CHUNK

NBUF = 8
WBUF = 3


def _make_kernel():
  mesh = plsc.VectorSubcoreMesh(core_axis_name="c", subcore_axis_name="s")

  @functools.partial(
      pl.kernel,
      mesh=mesh,
      compiler_params=pltpu.CompilerParams(
          use_tc_tiling_on_sc=False, needs_layout_passes=False),
      out_type=jax.ShapeDtypeStruct((T_LEN, 8, BLKS, 8, CHUNK), jnp.float32),
      scratch_types=[
          pltpu.VMEM((N_CHUNKS, CHUNK), jnp.int32),
          pltpu.VMEM((NBUF, CHUNK, D), jnp.float32),
          pltpu.VMEM((WBUF, 8, 8, CHUNK + 1), jnp.float32),
          pltpu.SemaphoreType.DMA((NBUF,)),
          pltpu.SemaphoreType.DMA((WBUF,)),
      ],
  )
  def emb_kernel(idx_hbm, table_hbm, out_hbm, idx_v, rows_v, tp_v, gsem, wsem):
    wid = lax.axis_index("s") * NC + lax.axis_index("c")
    pair0 = wid * N_CHUNKS

    pltpu.sync_copy(idx_hbm.at[wid], idx_v)

    lane = lax.iota(jnp.int32, 16)
    dlo_c = lax.rem(lane, 8)
    dhi_c = [2 * d0 + lane // 8 for d0 in range(4)]

    def gather(j, gb):
      return pltpu.make_async_copy(
          table_hbm.at[idx_v.at[j]], rows_v.at[gb], gsem.at[gb])

    def write(j, wb):
      q = pair0 + j
      t = (q // 256) * 8 + lax.rem(q, 8)
      blk = lax.rem(q // 8, BLKS)
      return pltpu.make_async_copy(
          tp_v.at[wb, :, :, pl.ds(0, CHUNK)], out_hbm.at[t, :, blk],
          wsem.at[wb])

    def transpose(gb, wb):
      rows = rows_v.at[gb]
      tp = tp_v.at[wb]

      def bbody(i, _):
        for k in range(2):
          b = 2 * i + k
          bv = jnp.full((16,), b, jnp.int32)
          for d0 in range(4):
            v = rows[b, pl.ds(16 * d0, 16)]
            plsc.store_scatter(tp, [dhi_c[d0], dlo_c, bv], v)
        return 0

      lax.fori_loop(0, CHUNK // 2, bbody, 0)

    def step(j, gb, wb, first, last):
      gather(j, gb).wait()
      if not first:
        write(j - WBUF, wb).wait()
      transpose(gb, wb)
      write(j, wb).start()
      if not last:
        gather(j + NBUF, gb).start()

    for gb in range(NBUF):
      gather(gb, gb).start()
    for j in range(NBUF):
      step(j, j % NBUF, j % WBUF, first=(j < WBUF), last=False)

    def body(i, _):
      for u in range(NBUF):
        j = i * NBUF + u
        step(j, u, j % WBUF, first=False, last=False)
      return 0

    lax.fori_loop(1, N_CHUNKS // NBUF - 1, body, 0)

    for j in range(N_CHUNKS - NBUF, N_CHUNKS):
      step(j, j % NBUF, j % WBUF, first=False, last=True)

    for j in range(N_CHUNKS - WBUF, N_CHUNKS):
      write(j, j % WBUF).wait()

  return emb_kernel


_emb = _make_kernel()


@jax.jit
def kernel(x, weight):
  idx = (x.astype(jnp.int32).reshape(BLKS, CHUNK, T_LEN // 8, 8)
         .transpose(2, 0, 3, 1).reshape(NW, N_CHUNKS, CHUNK))
  y = _emb(idx, weight)
  out = y.transpose(2, 4, 0, 1, 3).reshape(B, T_LEN, D)
  return out


# speedup vs baseline: 1.8393x; 1.0019x over previous
import functools

import jax
import jax.numpy as jnp
from jax import lax
from jax.experimental import pallas as pl
from jax.experimental.pallas import tpu as pltpu
from jax.experimental.pallas import tpu_sc as plsc

VOCAB = 1000000
D = 64
T_LEN = 200
B = 4096
B_TOTAL = B * T_LEN

NC = 2
NS = 16
NW = NC * NS

CHUNK = 128
PER_W = B_TOTAL // NW
N_CHUNKS = PER_W // CHUNK
BLKS = B // CHUNK

NBUF = 8
WBUF = 3


def _make_kernel():
  mesh = plsc.VectorSubcoreMesh(core_axis_name="c", subcore_axis_name="s")

  @functools.partial(
      pl.kernel,
      mesh=mesh,
      compiler_params=pltpu.CompilerParams(
          use_tc_tiling_on_sc=False, needs_layout_passes=False),
      out_type=jax.ShapeDtypeStruct((T_LEN, 8, BLKS, 8, CHUNK), jnp.float32),
      scratch_types=[
          pltpu.VMEM((N_CHUNKS, CHUNK), jnp.int32),
          pltpu.VMEM((NBUF, CHUNK, D), jnp.float32),
          pltpu.VMEM((WBUF, 8, 8, CHUNK + 1), jnp.float32),
          pltpu.SemaphoreType.DMA((NBUF,)),
          pltpu.SemaphoreType.DMA((WBUF,)),
      ],
  )
  def emb_kernel(idx_hbm, table_hbm, out_hbm, idx_v, rows_v, tp_v, gsem, wsem):
    wid = lax.axis_index("s") * NC + lax.axis_index("c")
    pair0 = wid * N_CHUNKS

    pltpu.sync_copy(idx_hbm.at[wid], idx_v)

    lane = lax.iota(jnp.int32, 16)
    dlo_c = lax.rem(lane, 8)
    dhi_c = [2 * d0 + lane // 8 for d0 in range(4)]

    def gather(j, gb):
      return pltpu.make_async_copy(
          table_hbm.at[idx_v.at[j]], rows_v.at[gb], gsem.at[gb])

    def write(j, wb):
      q = pair0 + j
      t = (q // 256) * 8 + lax.rem(q, 8)
      blk = lax.rem(q // 8, BLKS)
      return pltpu.make_async_copy(
          tp_v.at[wb, :, :, pl.ds(0, CHUNK)], out_hbm.at[t, :, blk],
          wsem.at[wb])

    def transpose(gb, wb):
      rows = rows_v.at[gb]
      tp = tp_v.at[wb]

      def bbody(i, _):
        for k in range(4):
          b = 4 * i + k
          bv = jnp.full((16,), b, jnp.int32)
          for d0 in range(4):
            v = rows[b, pl.ds(16 * d0, 16)]
            plsc.store_scatter(tp, [dhi_c[d0], dlo_c, bv], v)
        return 0

      lax.fori_loop(0, CHUNK // 4, bbody, 0)

    def step(j, gb, wb, first, last):
      gather(j, gb).wait()
      if not first:
        write(j - WBUF, wb).wait()
      transpose(gb, wb)
      write(j, wb).start()
      if not last:
        gather(j + NBUF, gb).start()

    for gb in range(NBUF):
      gather(gb, gb).start()
    for j in range(NBUF):
      step(j, j % NBUF, j % WBUF, first=(j < WBUF), last=False)

    def body(i, _):
      for u in range(NBUF):
        j = i * NBUF + u
        step(j, u, j % WBUF, first=False, last=False)
      return 0

    lax.fori_loop(1, N_CHUNKS // NBUF - 1, body, 0)

    for j in range(N_CHUNKS - NBUF, N_CHUNKS):
      step(j, j % NBUF, j % WBUF, first=False, last=True)

    for j in range(N_CHUNKS - WBUF, N_CHUNKS):
      write(j, j % WBUF).wait()

  return emb_kernel


_emb = _make_kernel()


@jax.jit
def kernel(x, weight):
  idx = (x.astype(jnp.int32).reshape(BLKS, CHUNK, T_LEN // 8, 8)
         .transpose(2, 0, 3, 1).reshape(NW, N_CHUNKS, CHUNK))
  y = _emb(idx, weight)
  out = y.transpose(2, 4, 0, 1, 3).reshape(B, T_LEN, D)
  return out
